# Initial kernel scaffold; baseline (speedup 1.0000x reference)
import jax
import jax.numpy as jnp
from jax import lax
from jax.experimental import pallas as pl
from jax.experimental.pallas import tpu as pltpu
from jax.experimental.pallas import tpu_sc as plsc

_EPS = 1e-5


def _sc_embed(ids_flat, pos_flat, comm_ids, word_table, community_table,
              pos_table, ln_w, ln_b, B, S, E):
    NC, NS = 2, 16
    NW = NC * NS
    ROWS_PER_TILE = B // NW
    NCHUNK = E // 16

    def body(ids_hbm, pos_ids_hbm, comm_ids_hbm, word_hbm, comm_hbm, pos_hbm,
             lnw_hbm, lnb_hbm, out_hbm,
             cidx_v, crows_v, widx_v, pidx_v, wrows_v, prows_v, obuf_v,
             lnw_v, lnb_v, sem1, sem2):
        wid = lax.axis_index("s") * NC + lax.axis_index("c")
        row0 = wid * ROWS_PER_TILE

        pltpu.sync_copy(comm_ids_hbm.at[pl.ds(row0, ROWS_PER_TILE)], cidx_v)
        pltpu.async_copy(comm_hbm.at[cidx_v], crows_v, sem1).wait()
        pltpu.sync_copy(lnw_hbm, lnw_v)
        pltpu.sync_copy(lnb_hbm, lnb_v)
        lnw = [lnw_v[pl.ds(16 * c, 16)] for c in range(NCHUNK)]
        lnb = [lnb_v[pl.ds(16 * c, 16)] for c in range(NCHUNK)]

        def row_body(r, carry):
            tok0 = pl.multiple_of((row0 + r) * S, 8)
            pltpu.sync_copy(ids_hbm.at[pl.ds(tok0, S)], widx_v)
            pltpu.sync_copy(pos_ids_hbm.at[pl.ds(tok0, S)], pidx_v)
            cps = []
            for lo, n in ((0, 128), (128, S - 128)):
                cps.append(pltpu.async_copy(
                    word_hbm.at[widx_v.at[pl.ds(lo, n)]],
                    wrows_v.at[pl.ds(lo, n)], sem1))
                cps.append(pltpu.async_copy(
                    pos_hbm.at[pidx_v.at[pl.ds(lo, n)]],
                    prows_v.at[pl.ds(lo, n)], sem2))
            for cp in cps:
                cp.wait()
            cm = [crows_v[r, pl.ds(16 * c, 16)] for c in range(NCHUNK)]

            def tok_body(t, tc):
                x = [wrows_v[t, pl.ds(16 * c, 16)]
                     + prows_v[t, pl.ds(16 * c, 16)] + cm[c]
                     for c in range(NCHUNK)]
                s1v = (x[0] + x[1]) + (x[2] + x[3])
                s2v = (x[0] * x[0] + x[1] * x[1]) + (x[2] * x[2] + x[3] * x[3])
                s1 = jnp.sum(s1v)
                s2 = jnp.sum(s2v)
                mean = s1 * (1.0 / E)
                var = s2 * (1.0 / E) - mean * mean
                v = var + _EPS
                i = lax.bitcast_convert_type(v, jnp.int32)
                i = jnp.int32(0x5F3759DF) - (i >> 1)
                y = lax.bitcast_convert_type(i, jnp.float32)
                h = v * 0.5
                y = y * (1.5 - h * y * y)
                y = y * (1.5 - h * y * y)
                y = y * (1.5 - h * y * y)
                for c in range(NCHUNK):
                    obuf_v[t, pl.ds(16 * c, 16)] = (
                        (x[c] - mean) * y * lnw[c] + lnb[c])
                return tc
            lax.fori_loop(0, S, tok_body, 0)
            pltpu.sync_copy(obuf_v, out_hbm.at[pl.ds(tok0, S)])
            return carry

        lax.fori_loop(0, ROWS_PER_TILE, row_body, 0)

    mesh = plsc.VectorSubcoreMesh(core_axis_name="c", subcore_axis_name="s")
    fn = pl.kernel(
        body,
        out_type=jax.ShapeDtypeStruct((B * S, E), jnp.float32),
        mesh=mesh,
        scratch_types=[
            pltpu.VMEM((ROWS_PER_TILE,), jnp.int32),
            pltpu.VMEM((ROWS_PER_TILE, E), jnp.float32),
            pltpu.VMEM((S,), jnp.int32),
            pltpu.VMEM((S,), jnp.int32),
            pltpu.VMEM((S, E), jnp.float32),
            pltpu.VMEM((S, E), jnp.float32),
            pltpu.VMEM((S, E), jnp.float32),
            pltpu.VMEM((E,), jnp.float32),
            pltpu.VMEM((E,), jnp.float32),
            pltpu.SemaphoreType.DMA,
            pltpu.SemaphoreType.DMA,
        ],
    )
    return fn(ids_flat, pos_flat, comm_ids, word_table, community_table,
              pos_table, ln_w, ln_b)


def kernel(input_ids, community_ids, position_ids, word_table,
           community_table, pos_table, ln_w, ln_b):
    B, S = input_ids.shape
    E = word_table.shape[1]
    ids_flat = input_ids.reshape(-1).astype(jnp.int32)
    pos_flat = position_ids.reshape(-1).astype(jnp.int32)
    comm = community_ids.astype(jnp.int32)
    out = _sc_embed(ids_flat, pos_flat, comm, word_table, community_table,
                    pos_table, ln_w, ln_b, B, S, E)
    return out.reshape(B, S, E)


# speedup vs baseline: 2.0115x; 2.0115x over previous
import jax
import jax.numpy as jnp
from jax import lax
from jax.experimental import pallas as pl
from jax.experimental.pallas import tpu as pltpu
from jax.experimental.pallas import tpu_sc as plsc

_EPS = 1e-5


def _sc_embed(ids_flat, pos_flat, comm_ids, word_table, community_table,
              pos_table, ln_w, ln_b, B, S, E):
    NC, NS = 2, 16
    NW = NC * NS
    ROWS_PER_TILE = B // NW
    NCHUNK = E // 16

    def body(ids_hbm, pos_ids_hbm, comm_ids_hbm, word_hbm, comm_hbm, pos_hbm,
             lnw_hbm, lnb_hbm, out_hbm,
             cidx_v, crows_v, widx_v, pidx_v, wrows_v, prows_v, obuf_v,
             lnw_v, lnb_v, sem1, sem2):
        wid = lax.axis_index("s") * NC + lax.axis_index("c")
        row0 = wid * ROWS_PER_TILE

        pltpu.sync_copy(comm_ids_hbm.at[pl.ds(row0, ROWS_PER_TILE)], cidx_v)
        pltpu.async_copy(comm_hbm.at[cidx_v], crows_v, sem1).wait()
        pltpu.sync_copy(lnw_hbm, lnw_v)
        pltpu.sync_copy(lnb_hbm, lnb_v)
        lnw = [lnw_v[pl.ds(16 * c, 16)] for c in range(NCHUNK)]
        lnb = [lnb_v[pl.ds(16 * c, 16)] for c in range(NCHUNK)]

        lane = lax.iota(jnp.int32, 16)
        bfly = [lane ^ jnp.int32(stride) for stride in (8, 4, 2, 1)]

        def hsum(vv):
            for idx in bfly:
                vv = vv + vv.at[idx].get(mode="promise_in_bounds")
            return vv

        def row_body(r, carry):
            tok0 = pl.multiple_of((row0 + r) * S, 8)
            pltpu.sync_copy(ids_hbm.at[pl.ds(tok0, S)], widx_v)
            pltpu.sync_copy(pos_ids_hbm.at[pl.ds(tok0, S)], pidx_v)
            cps = []
            for lo, n in ((0, 128), (128, S - 128)):
                cps.append(pltpu.async_copy(
                    word_hbm.at[widx_v.at[pl.ds(lo, n)]],
                    wrows_v.at[pl.ds(lo, n)], sem1))
                cps.append(pltpu.async_copy(
                    pos_hbm.at[pidx_v.at[pl.ds(lo, n)]],
                    prows_v.at[pl.ds(lo, n)], sem2))
            for cp in cps:
                cp.wait()
            cm = [crows_v[r, pl.ds(16 * c, 16)] for c in range(NCHUNK)]

            def tok_body(t, tc):
                x = [wrows_v[t, pl.ds(16 * c, 16)]
                     + prows_v[t, pl.ds(16 * c, 16)] + cm[c]
                     for c in range(NCHUNK)]
                s1v = (x[0] + x[1]) + (x[2] + x[3])
                s2v = (x[0] * x[0] + x[1] * x[1]) + (x[2] * x[2] + x[3] * x[3])
                mean = hsum(s1v) * (1.0 / E)
                var = hsum(s2v) * (1.0 / E) - mean * mean
                v = var + _EPS
                i = lax.bitcast_convert_type(v, jnp.int32)
                i = jnp.int32(0x5F3759DF) - (i >> 1)
                y = lax.bitcast_convert_type(i, jnp.float32)
                h = v * 0.5
                y = y * (1.5 - h * y * y)
                y = y * (1.5 - h * y * y)
                y = y * (1.5 - h * y * y)
                for c in range(NCHUNK):
                    obuf_v[t, pl.ds(16 * c, 16)] = (
                        (x[c] - mean) * y * lnw[c] + lnb[c])
                return tc
            lax.fori_loop(0, S, tok_body, 0)
            pltpu.sync_copy(obuf_v, out_hbm.at[pl.ds(tok0, S)])
            return carry

        lax.fori_loop(0, ROWS_PER_TILE, row_body, 0)

    mesh = plsc.VectorSubcoreMesh(core_axis_name="c", subcore_axis_name="s")
    fn = pl.kernel(
        body,
        out_type=jax.ShapeDtypeStruct((B * S, E), jnp.float32),
        mesh=mesh,
        compiler_params=pltpu.CompilerParams(use_tc_tiling_on_sc=False),
        scratch_types=[
            pltpu.VMEM((ROWS_PER_TILE,), jnp.int32),
            pltpu.VMEM((ROWS_PER_TILE, E), jnp.float32),
            pltpu.VMEM((S,), jnp.int32),
            pltpu.VMEM((S,), jnp.int32),
            pltpu.VMEM((S, E), jnp.float32),
            pltpu.VMEM((S, E), jnp.float32),
            pltpu.VMEM((S, E), jnp.float32),
            pltpu.VMEM((E,), jnp.float32),
            pltpu.VMEM((E,), jnp.float32),
            pltpu.SemaphoreType.DMA,
            pltpu.SemaphoreType.DMA,
        ],
    )
    return fn(ids_flat, pos_flat, comm_ids, word_table, community_table,
              pos_table, ln_w, ln_b)


def kernel(input_ids, community_ids, position_ids, word_table,
           community_table, pos_table, ln_w, ln_b):
    B, S = input_ids.shape
    E = word_table.shape[1]
    ids_flat = input_ids.reshape(-1).astype(jnp.int32)
    pos_flat = position_ids.reshape(-1).astype(jnp.int32)
    comm = community_ids.astype(jnp.int32)
    out = _sc_embed(ids_flat, pos_flat, comm, word_table, community_table,
                    pos_table, ln_w, ln_b, B, S, E)
    return out.reshape(B, S, E)


# speedup vs baseline: 2.0343x; 1.0113x over previous
import jax
import jax.numpy as jnp
from jax import lax
from jax.experimental import pallas as pl
from jax.experimental.pallas import tpu as pltpu
from jax.experimental.pallas import tpu_sc as plsc

_EPS = 1e-5


def _sc_embed(ids_flat, pos_flat, comm_ids, word_table, community_table,
              pos_table, ln_w, ln_b, B, S, E):
    NC, NS = 2, 16
    NW = NC * NS
    ROWS_PER_TILE = B // NW
    NCHUNK = E // 16

    def body(ids_hbm, pos_ids_hbm, comm_ids_hbm, word_hbm, comm_hbm, pos_hbm,
             lnw_hbm, lnb_hbm, out_hbm,
             cidx_v, crows_v, widx_v, pidx_v, wrows_v, prows_v, obuf_v,
             lnw_v, lnb_v, sem1, sem2):
        wid = lax.axis_index("s") * NC + lax.axis_index("c")
        row0 = wid * ROWS_PER_TILE

        pltpu.sync_copy(comm_ids_hbm.at[pl.ds(row0, ROWS_PER_TILE)], cidx_v)
        pltpu.async_copy(comm_hbm.at[cidx_v], crows_v, sem1).wait()
        pltpu.sync_copy(lnw_hbm, lnw_v)
        pltpu.sync_copy(lnb_hbm, lnb_v)
        lnw = [lnw_v[pl.ds(16 * c, 16)] for c in range(NCHUNK)]
        lnb = [lnb_v[pl.ds(16 * c, 16)] for c in range(NCHUNK)]

        lane = lax.iota(jnp.int32, 16)
        bfly = [lane ^ jnp.int32(stride) for stride in (8, 4, 2, 1)]

        def hsum(vv):
            for idx in bfly:
                vv = vv + vv.at[idx].get(mode="promise_in_bounds")
            return vv

        def row_body(r, carry):
            tok0 = pl.multiple_of((row0 + r) * S, 8)
            pltpu.sync_copy(ids_hbm.at[pl.ds(tok0, S)], widx_v)
            pltpu.sync_copy(pos_ids_hbm.at[pl.ds(tok0, S)], pidx_v)
            cps = []
            for lo, n in ((0, 128), (128, S - 128)):
                cps.append(pltpu.async_copy(
                    word_hbm.at[widx_v.at[pl.ds(lo, n)]],
                    wrows_v.at[pl.ds(lo, n)], sem1))
                cps.append(pltpu.async_copy(
                    pos_hbm.at[pidx_v.at[pl.ds(lo, n)]],
                    prows_v.at[pl.ds(lo, n)], sem2))
            for cp in cps:
                cp.wait()
            cm = [crows_v[r, pl.ds(16 * c, 16)] for c in range(NCHUNK)]

            @plsc.parallel_loop(0, S, step=1, unroll=4)
            def _(t):
                x = [wrows_v[t, pl.ds(16 * c, 16)]
                     + prows_v[t, pl.ds(16 * c, 16)] + cm[c]
                     for c in range(NCHUNK)]
                s1v = (x[0] + x[1]) + (x[2] + x[3])
                s2v = (x[0] * x[0] + x[1] * x[1]) + (x[2] * x[2] + x[3] * x[3])
                mean = hsum(s1v) * (1.0 / E)
                var = hsum(s2v) * (1.0 / E) - mean * mean
                v = var + _EPS
                i = lax.bitcast_convert_type(v, jnp.int32)
                i = jnp.int32(0x5F3759DF) - (i >> 1)
                y = lax.bitcast_convert_type(i, jnp.float32)
                h = v * 0.5
                y = y * (1.5 - h * y * y)
                y = y * (1.5 - h * y * y)
                for c in range(NCHUNK):
                    obuf_v[t, pl.ds(16 * c, 16)] = (
                        (x[c] - mean) * y * lnw[c] + lnb[c])
            pltpu.sync_copy(obuf_v, out_hbm.at[pl.ds(tok0, S)])
            return carry

        lax.fori_loop(0, ROWS_PER_TILE, row_body, 0)

    mesh = plsc.VectorSubcoreMesh(core_axis_name="c", subcore_axis_name="s")
    fn = pl.kernel(
        body,
        out_type=jax.ShapeDtypeStruct((B * S, E), jnp.float32),
        mesh=mesh,
        compiler_params=pltpu.CompilerParams(use_tc_tiling_on_sc=False),
        scratch_types=[
            pltpu.VMEM((ROWS_PER_TILE,), jnp.int32),
            pltpu.VMEM((ROWS_PER_TILE, E), jnp.float32),
            pltpu.VMEM((S,), jnp.int32),
            pltpu.VMEM((S,), jnp.int32),
            pltpu.VMEM((S, E), jnp.float32),
            pltpu.VMEM((S, E), jnp.float32),
            pltpu.VMEM((S, E), jnp.float32),
            pltpu.VMEM((E,), jnp.float32),
            pltpu.VMEM((E,), jnp.float32),
            pltpu.SemaphoreType.DMA,
            pltpu.SemaphoreType.DMA,
        ],
    )
    return fn(ids_flat, pos_flat, comm_ids, word_table, community_table,
              pos_table, ln_w, ln_b)


def kernel(input_ids, community_ids, position_ids, word_table,
           community_table, pos_table, ln_w, ln_b):
    B, S = input_ids.shape
    E = word_table.shape[1]
    ids_flat = input_ids.reshape(-1).astype(jnp.int32)
    pos_flat = position_ids.reshape(-1).astype(jnp.int32)
    comm = community_ids.astype(jnp.int32)
    out = _sc_embed(ids_flat, pos_flat, comm, word_table, community_table,
                    pos_table, ln_w, ln_b, B, S, E)
    return out.reshape(B, S, E)
